# baseline (speedup 1.0000x reference)
import functools
import math

import jax
import jax.numpy as jnp
from jax import lax
from jax.experimental import pallas as pl
from jax.experimental.pallas import tpu as pltpu
from jax.experimental.pallas import tpu_sc as plsc

N = 10000
E = 160000
IN = 256
HID = 64
HEADS = 10
OUT = 256
D1 = HEADS * HID

NC = 2
NS = 16
NW = NC * NS
G = 96
GPW = G // NW
GN = 112
NPAD = G * GN
CAP = 2176
BA = 24
BV = 24

_MESH = dict(core_axis_name="c", subcore_axis_name="s")

_GDN = lax.GatherDimensionNumbers(
    offset_dims=(), collapsed_slice_dims=(0,), start_index_map=(0,))


def _dyn_gather(x, idx):
    return lax.gather(x, idx[:, None], _GDN, (1,),
                      mode=lax.GatherScatterMode.PROMISE_IN_BOUNDS)


def _wid():
    return lax.axis_index("s") * NC + lax.axis_index("c")


_BM = 512
_GRID = (N + _BM - 1) // _BM


def _proj_body(x_ref, wq, bq, wk, bk, wv, bv, ws, bs, oq, ok, ov, os_):
    xb = x_ref[...]
    oq[...] = jnp.dot(xb, wq[...], preferred_element_type=jnp.float32) + bq[...]
    ok[...] = jnp.dot(xb, wk[...], preferred_element_type=jnp.float32) + bk[...]
    ov[...] = jnp.dot(xb, wv[...], preferred_element_type=jnp.float32) + bv[...]
    os_[...] = jnp.dot(xb, ws[...], preferred_element_type=jnp.float32) + bs[...]


def _dense_proj(x, Wq, bq, Wk, bk, Wv, bv, Ws, bs):
    din, dout = Wq.shape
    nrows = x.shape[0]
    wspec = pl.BlockSpec((din, dout), lambda i: (0, 0))
    bspec = pl.BlockSpec((1, dout), lambda i: (0, 0))
    return pl.pallas_call(
        _proj_body,
        grid=(_GRID,),
        in_specs=[pl.BlockSpec((_BM, din), lambda i: (i, 0)),
                  wspec, bspec, wspec, bspec, wspec, bspec, wspec, bspec],
        out_specs=[pl.BlockSpec((_BM, dout), lambda i: (i, 0))] * 4,
        out_shape=[jax.ShapeDtypeStruct((N, dout), jnp.float32)] * 4,
    )(x, Wq, bq.reshape(1, dout), Wk, bk.reshape(1, dout),
      Wv, bv.reshape(1, dout), Ws, bs.reshape(1, dout))


def _ln_relu(t, g, be):
    m = jnp.mean(t, -1, keepdims=True)
    c = t - m
    var = jnp.mean(c * c, -1, keepdims=True)
    hn = c * lax.rsqrt(var + 1e-5) * g + be
    return jnp.maximum(hn, 0.0)


def _mid_body(agg, s, g, be, wq, bq, wk, bk, wv, bv, ws, bs, oq, ok, ov, os_):
    h = _ln_relu(agg[...] + s[...], g[...], be[...])
    oq[...] = jnp.dot(h, wq[...], preferred_element_type=jnp.float32) + bq[...]
    ok[...] = jnp.dot(h, wk[...], preferred_element_type=jnp.float32) + bk[...]
    ov[...] = jnp.dot(h, wv[...], preferred_element_type=jnp.float32) + bv[...]
    os_[...] = jnp.dot(h, ws[...], preferred_element_type=jnp.float32) + bs[...]


def _dense_mid(agg, s, g, be, Wq, bq, Wk, bk, Wv, bv, Ws, bs):
    din, dout = Wq.shape
    wspec = pl.BlockSpec((din, dout), lambda i: (0, 0))
    bspec = pl.BlockSpec((1, dout), lambda i: (0, 0))
    gspec = pl.BlockSpec((1, din), lambda i: (0, 0))
    return pl.pallas_call(
        _mid_body,
        grid=(_GRID,),
        in_specs=[pl.BlockSpec((_BM, din), lambda i: (i, 0)),
                  pl.BlockSpec((_BM, din), lambda i: (i, 0)),
                  gspec, gspec,
                  wspec, bspec, wspec, bspec, wspec, bspec, wspec, bspec],
        out_specs=[pl.BlockSpec((_BM, dout), lambda i: (i, 0))] * 4,
        out_shape=[jax.ShapeDtypeStruct((N, dout), jnp.float32)] * 4,
    )(agg, s, g.reshape(1, din), be.reshape(1, din),
      Wq, bq.reshape(1, dout), Wk, bk.reshape(1, dout),
      Wv, bv.reshape(1, dout), Ws, bs.reshape(1, dout))


def _fin_body(agg, s, g, be, o):
    o[...] = _ln_relu(agg[...] + s[...], g[...], be[...])


def _dense_fin(agg, s, g, be):
    d = s.shape[1]
    gspec = pl.BlockSpec((1, d), lambda i: (0, 0))
    return pl.pallas_call(
        _fin_body,
        grid=(_GRID,),
        in_specs=[pl.BlockSpec((_BM, d), lambda i: (i, 0)),
                  pl.BlockSpec((_BM, d), lambda i: (i, 0)), gspec, gspec],
        out_specs=pl.BlockSpec((_BM, d), lambda i: (i, 0)),
        out_shape=jax.ShapeDtypeStruct((N, d), jnp.float32),
    )(agg, s, g.reshape(1, d), be.reshape(1, d))


_CH = 1280


def _sc_compact(src, dst):
    mesh = plsc.VectorSubcoreMesh(**_MESH)

    @functools.partial(
        pl.kernel,
        out_type=(jax.ShapeDtypeStruct((G * CAP,), jnp.int32),
                  jax.ShapeDtypeStruct((G * CAP,), jnp.int32),
                  jax.ShapeDtypeStruct((G * 128,), jnp.int32)),
        mesh=mesh,
        compiler_params=pltpu.CompilerParams(needs_layout_passes=False),
        scratch_types=[
            pltpu.VMEM((_CH,), jnp.int32),
            pltpu.VMEM((_CH,), jnp.int32),
            pltpu.VMEM((CAP,), jnp.int32),
            pltpu.VMEM((CAP,), jnp.int32),
            pltpu.VMEM((CAP,), jnp.int32),
            pltpu.VMEM((CAP,), jnp.int32),
            pltpu.VMEM((CAP,), jnp.int32),
            pltpu.VMEM((CAP,), jnp.int32),
            pltpu.VMEM((128,), jnp.int32),
        ],
    )
    def kern(src_h, dst_h, os_h, od_h, cnt_h, sb, db,
             ls0, ls1, ls2, ld0, ld1, ld2, cb):
        ls = [ls0, ls1, ls2]
        ld = [ld0, ld1, ld2]
        wid = _wid()
        lane = lax.broadcasted_iota(jnp.int32, (16,), 0)
        zero16 = jnp.zeros((16,), jnp.int32)

        def zbody(i, _):
            for g_i in range(GPW):
                ls[g_i][pl.ds(i * 16, 16)] = zero16
                ld[g_i][pl.ds(i * 16, 16)] = zero16
            return 0
        lax.fori_loop(0, CAP // 16, zbody, 0)

        glo0 = wid * GPW * GN

        def chunk(ci, curs):
            base = ci * _CH
            pltpu.sync_copy(src_h.at[pl.ds(base, _CH)], sb)
            pltpu.sync_copy(dst_h.at[pl.ds(base, _CH)], db)

            @plsc.parallel_loop(0, _CH // 16, 1, unroll=2, carry=curs)
            def step(i, curs):
                sv = sb[pl.ds(i * 16, 16)]
                dv = db[pl.ds(i * 16, 16)]
                out = []
                for g_i in range(GPW):
                    lo = glo0 + g_i * GN
                    m = (dv >= lo) & (dv < lo + GN)
                    pos = curs[g_i] + plsc.cumsum(jnp.where(m, 1, 0)) - 1
                    mm = m & (pos < CAP)
                    plsc.store_scatter(ls[g_i], [pos], sv, mask=mm)
                    plsc.store_scatter(ld[g_i], [pos], dv, mask=mm)
                    out.append(curs[g_i] + plsc.all_reduce_population_count(m))
                return tuple(out)

            return step

        curs = lax.fori_loop(0, E // _CH, chunk,
                             tuple(zero16 for _ in range(GPW)))

        for g_i in range(GPW):
            grp = wid * GPW + g_i
            pltpu.sync_copy(ls[g_i], os_h.at[pl.ds(grp * CAP, CAP)])
            pltpu.sync_copy(ld[g_i], od_h.at[pl.ds(grp * CAP, CAP)])
            cb[pl.ds(0, 16)] = curs[g_i]
            pltpu.sync_copy(cb, cnt_h.at[pl.ds(grp * 128, 128)])

    return kern(src, dst)


def _sc_edge_attn(q, k, osrc, odst, cnts, H, C):
    D = H * C
    mesh = plsc.VectorSubcoreMesh(**_MESH)
    inv = 1.0 / math.sqrt(C)

    @functools.partial(
        pl.kernel,
        out_type=jax.ShapeDtypeStruct((G * CAP, 16), jnp.float32),
        mesh=mesh,
        compiler_params=pltpu.CompilerParams(needs_layout_passes=False),
        scratch_types=[
            pltpu.VMEM((CAP + 32,), jnp.int32),
            pltpu.VMEM((CAP + 32,), jnp.int32),
            pltpu.VMEM((128,), jnp.int32),
            pltpu.VMEM((GN, D), jnp.float32),
            pltpu.VMEM((BA, D), jnp.float32),
            pltpu.VMEM((BA, D), jnp.float32),
            pltpu.VMEM((BA, 16), jnp.float32),
            pltpu.SemaphoreType.DMA,
            pltpu.SemaphoreType.DMA,
        ],
    )
    def kern(q_h, k_h, os_h, od_h, cnt_h, ea_h, ls, ld, cb,
             qg, kb0, kb1, eab, smk0, smk1):
        wid = _wid()
        lane = lax.broadcasted_iota(jnp.int32, (16,), 0)
        zi = jnp.zeros((16,), jnp.int32)
        for t in range(2):
            ls[pl.ds(CAP + t * 16, 16)] = zi
            ld[pl.ds(CAP + t * 16, 16)] = zi
        perms = [lane ^ (1 << t) for t in range(4)]
        bufs = [(kb0, smk0), (kb1, smk1)]

        for g_i in range(GPW):
            grp = wid * GPW + g_i
            glo = grp * GN
            pltpu.sync_copy(os_h.at[pl.ds(grp * CAP, CAP)],
                            ls.at[pl.ds(0, CAP)])
            pltpu.sync_copy(od_h.at[pl.ds(grp * CAP, CAP)],
                            ld.at[pl.ds(0, CAP)])
            pltpu.sync_copy(cnt_h.at[pl.ds(grp * 128, 128)], cb)
            pltpu.sync_copy(q_h.at[pl.ds(glo, GN)], qg)
            cnt = jnp.minimum(cb[pl.ds(0, 16)][0], CAP)
            nch = (cnt + BA - 1) // BA

            def issue(j, b):
                kb, smk = bufs[b]
                pltpu.async_copy(k_h.at[ls.at[pl.ds(j * BA, BA)]], kb, smk)

            @pl.when(nch > 0)
            def _():
                issue(0, 0)

            def duo(jo, _):
                for b in range(2):
                    j = 2 * jo + b

                    @pl.when(j < nch)
                    def _():
                        @pl.when(j + 1 < nch)
                        def _():
                            issue(j + 1, 1 - b)
                        kb, smk = bufs[b]
                        pltpu.make_async_copy(
                            k_h.at[pl.ds(0, BA)], kb, smk).wait()
                        dvl0 = ld[pl.ds(j * BA, 16)]
                        dvl1 = ld[pl.ds(j * BA + 8, 16)]

                        @plsc.parallel_loop(0, BA, 1, unroll=3)
                        def edge(e):
                            lsp = jnp.broadcast_to(e & 7, (16,)) \
                                .astype(jnp.int32)
                            dsel = jnp.where(e < 16, _dyn_gather(dvl0, lsp + (e & 8)),
                                             _dyn_gather(dvl1, lsp + 8))
                            r0 = dsel[0] - glo
                            r = jnp.clip(r0, 0, GN - 1)
                            al = jnp.zeros((16,), jnp.float32)
                            for h in range(H):
                                acc = jnp.zeros((16,), jnp.float32)
                                for i in range(C // 16):
                                    off = h * C + i * 16
                                    acc = acc + (qg[r, pl.ds(off, 16)] *
                                                 kb[e, pl.ds(off, 16)])
                                for t in range(4):
                                    acc = acc + _dyn_gather(acc, perms[t])
                                al = jnp.where(lane == h, acc, al)
                            eab[e, :] = jnp.exp(al * inv)

                        pltpu.sync_copy(
                            eab, ea_h.at[pl.ds(grp * CAP + j * BA, BA)])
                return 0

            lax.fori_loop(0, (nch + 1) // 2, duo, 0)

    return kern(q, k, osrc, odst, cnts)


def _sc_edge_agg(v, ea, osrc, odst, cnts, H, C):
    D = H * C
    NV = D // 16
    mesh = plsc.VectorSubcoreMesh(**_MESH)

    @functools.partial(
        pl.kernel,
        out_type=jax.ShapeDtypeStruct((NPAD, D), jnp.float32),
        mesh=mesh,
        compiler_params=pltpu.CompilerParams(needs_layout_passes=False),
        scratch_types=[
            pltpu.VMEM((CAP + 32,), jnp.int32),
            pltpu.VMEM((CAP + 32,), jnp.int32),
            pltpu.VMEM((128,), jnp.int32),
            pltpu.VMEM((BV, D), jnp.float32),
            pltpu.VMEM((BV, D), jnp.float32),
            pltpu.VMEM((BV, 16), jnp.float32),
            pltpu.VMEM((BV, 16), jnp.float32),
            pltpu.VMEM((GN, D), jnp.float32),
            pltpu.VMEM((GN, 16), jnp.float32),
            pltpu.SemaphoreType.DMA,
            pltpu.SemaphoreType.DMA,
            pltpu.SemaphoreType.DMA,
            pltpu.SemaphoreType.DMA,
        ],
    )
    def kern(v_h, ea_h, os_h, od_h, cnt_h, agg_h,
             ls, ld, cb, vb0, vb1, eb0, eb1, acc, den,
             smv0, smv1, sme0, sme1):
        bufs = [(vb0, eb0, smv0, sme0), (vb1, eb1, smv1, sme1)]
        wid = _wid()
        zi = jnp.zeros((16,), jnp.int32)
        for t in range(2):
            ls[pl.ds(CAP + t * 16, 16)] = zi
            ld[pl.ds(CAP + t * 16, 16)] = zi
        lane = lax.broadcasted_iota(jnp.int32, (16,), 0)
        hmask = jnp.where(lane < H, 1.0, 0.0).astype(jnp.float32)
        zerov = jnp.zeros((16,), jnp.float32)
        hidx = [jnp.full((16,), h, jnp.int32) for h in range(H)]

        for g_i in range(GPW):
            grp = wid * GPW + g_i
            glo = grp * GN
            pltpu.sync_copy(os_h.at[pl.ds(grp * CAP, CAP)],
                            ls.at[pl.ds(0, CAP)])
            pltpu.sync_copy(od_h.at[pl.ds(grp * CAP, CAP)],
                            ld.at[pl.ds(0, CAP)])
            pltpu.sync_copy(cnt_h.at[pl.ds(grp * 128, 128)], cb)
            cnt = jnp.minimum(cb[pl.ds(0, 16)][0], CAP)

            @plsc.parallel_loop(0, GN, 1, unroll=1)
            def zbody(r):
                for i in range(NV):
                    acc[r, pl.ds(i * 16, 16)] = zerov
                den[r, :] = zerov

            nch = (cnt + BV - 1) // BV

            def issue(j, b):
                vb, eb, smv, sme = bufs[b]
                pltpu.async_copy(v_h.at[ls.at[pl.ds(j * BV, BV)]], vb, smv)
                pltpu.async_copy(ea_h.at[pl.ds(grp * CAP + j * BV, BV)],
                                 eb, sme)

            @pl.when(nch > 0)
            def _():
                issue(0, 0)

            def duo(jo, _):
                for pb in range(2):
                    ci = 2 * jo + pb

                    @pl.when(ci < nch)
                    def _():
                        @pl.when(ci + 1 < nch)
                        def _():
                            issue(ci + 1, 1 - pb)
                        vb, eb, smv, sme = bufs[pb]
                        pltpu.make_async_copy(
                            v_h.at[pl.ds(0, BV)], vb, smv).wait()
                        pltpu.make_async_copy(
                            ea_h.at[pl.ds(0, BV)], eb, sme).wait()
                        b = ci * BV
                        nn = jnp.minimum(BV, cnt - b)
                        dv0 = ld[pl.ds(b, 16)]
                        dv1 = ld[pl.ds(b + 8, 16)]

                        @plsc.parallel_loop(0, nn, 1, unroll=1)
                        def edge(e):
                            lsp = jnp.broadcast_to(e & 7, (16,)) \
                                .astype(jnp.int32)
                            dsel = jnp.where(
                                e < 16, _dyn_gather(dv0, lsp + (e & 8)),
                                _dyn_gather(dv1, lsp + 8))
                            r = dsel[0] - glo
                            eav = eb[e, :]
                            plsc.addupdate(den.at[r], eav * hmask)
                            esp = jnp.broadcast_to(e, (16,)).astype(jnp.int32)
                            for h in range(H):
                                ehv = plsc.load_gather(eb, [esp, hidx[h]])
                                for i in range(C // 16):
                                    off = h * C + i * 16
                                    plsc.addupdate(
                                        acc.at[r, pl.ds(off, 16)],
                                        ehv * vb[e, pl.ds(off, 16)])
                return 0

            lax.fori_loop(0, (nch + 1) // 2, duo, 0)

            @plsc.parallel_loop(0, GN, 1, unroll=1)
            def fin(r):
                rv = 1.0 / (den[r, :] + 1e-16)
                for h in range(H):
                    rhv = _dyn_gather(rv, hidx[h])
                    for i in range(C // 16):
                        off = h * C + i * 16
                        acc[r, pl.ds(off, 16)] = acc[r, pl.ds(off, 16)] * rhv

            pltpu.sync_copy(acc, agg_h.at[pl.ds(glo, GN)])

    return kern(v, ea, osrc, odst, cnts)


def kernel(x, edge_index, Wq1, bq1, Wk1, bk1, Wv1, bv1, Ws1, bs1, g1, be1,
           Wq2, bq2, Wk2, bk2, Wv2, bv2, Ws2, bs2, g2, be2):
    src = edge_index[0].astype(jnp.int32)
    dst = edge_index[1].astype(jnp.int32)

    q1, k1, v1, s1 = _dense_proj(x, Wq1, bq1, Wk1, bk1, Wv1, bv1, Ws1, bs1)
    osrc, odst, cnts = _sc_compact(src, dst)
    ea1 = _sc_edge_attn(q1, k1, osrc, odst, cnts, HEADS, HID)
    agg1 = _sc_edge_agg(v1, ea1, osrc, odst, cnts, HEADS, HID)

    q2, k2, v2, s2 = _dense_mid(agg1[:N], s1, g1, be1,
                                Wq2, bq2, Wk2, bk2, Wv2, bv2, Ws2, bs2)
    ea2 = _sc_edge_attn(q2, k2, osrc, odst, cnts, 1, OUT)
    agg2 = _sc_edge_agg(v2, ea2, osrc, odst, cnts, 1, OUT)

    return _dense_fin(agg2[:N], s2, g2, be2)


# speedup vs baseline: 1.2122x; 1.2122x over previous
import functools
import math

import jax
import jax.numpy as jnp
from jax import lax
from jax.experimental import pallas as pl
from jax.experimental.pallas import tpu as pltpu
from jax.experimental.pallas import tpu_sc as plsc

N = 10000
E = 160000
IN = 256
HID = 64
HEADS = 10
OUT = 256
D1 = HEADS * HID

NC = 2
NS = 16
NW = NC * NS
G = 96
GPW = G // NW
GN = 112
NPAD = G * GN
CAP = 2176
BA = 24
BV = 24

_MESH = dict(core_axis_name="c", subcore_axis_name="s")

_GDN = lax.GatherDimensionNumbers(
    offset_dims=(), collapsed_slice_dims=(0,), start_index_map=(0,))


def _dyn_gather(x, idx):
    return lax.gather(x, idx[:, None], _GDN, (1,),
                      mode=lax.GatherScatterMode.PROMISE_IN_BOUNDS)


def _wid():
    return lax.axis_index("s") * NC + lax.axis_index("c")


_BM = 512
_GRID = (N + _BM - 1) // _BM


def _proj_body(x_ref, wq, bq, wk, bk, wv, bv, ws, bs, oq, ok, ov, os_):
    xb = x_ref[...]
    oq[...] = jnp.dot(xb, wq[...], preferred_element_type=jnp.float32) + bq[...]
    ok[...] = jnp.dot(xb, wk[...], preferred_element_type=jnp.float32) + bk[...]
    ov[...] = jnp.dot(xb, wv[...], preferred_element_type=jnp.float32) + bv[...]
    os_[...] = jnp.dot(xb, ws[...], preferred_element_type=jnp.float32) + bs[...]


def _dense_proj(x, Wq, bq, Wk, bk, Wv, bv, Ws, bs):
    din, dout = Wq.shape
    nrows = x.shape[0]
    wspec = pl.BlockSpec((din, dout), lambda i: (0, 0))
    bspec = pl.BlockSpec((1, dout), lambda i: (0, 0))
    return pl.pallas_call(
        _proj_body,
        grid=(_GRID,),
        in_specs=[pl.BlockSpec((_BM, din), lambda i: (i, 0)),
                  wspec, bspec, wspec, bspec, wspec, bspec, wspec, bspec],
        out_specs=[pl.BlockSpec((_BM, dout), lambda i: (i, 0))] * 4,
        out_shape=[jax.ShapeDtypeStruct((N, dout), jnp.float32)] * 4,
    )(x, Wq, bq.reshape(1, dout), Wk, bk.reshape(1, dout),
      Wv, bv.reshape(1, dout), Ws, bs.reshape(1, dout))


def _ln_relu(t, g, be):
    m = jnp.mean(t, -1, keepdims=True)
    c = t - m
    var = jnp.mean(c * c, -1, keepdims=True)
    hn = c * lax.rsqrt(var + 1e-5) * g + be
    return jnp.maximum(hn, 0.0)


def _mid_body(agg, s, g, be, wq, bq, wk, bk, wv, bv, ws, bs, oq, ok, ov, os_):
    h = _ln_relu(agg[...] + s[...], g[...], be[...])
    oq[...] = jnp.dot(h, wq[...], preferred_element_type=jnp.float32) + bq[...]
    ok[...] = jnp.dot(h, wk[...], preferred_element_type=jnp.float32) + bk[...]
    ov[...] = jnp.dot(h, wv[...], preferred_element_type=jnp.float32) + bv[...]
    os_[...] = jnp.dot(h, ws[...], preferred_element_type=jnp.float32) + bs[...]


def _dense_mid(agg, s, g, be, Wq, bq, Wk, bk, Wv, bv, Ws, bs):
    din, dout = Wq.shape
    wspec = pl.BlockSpec((din, dout), lambda i: (0, 0))
    bspec = pl.BlockSpec((1, dout), lambda i: (0, 0))
    gspec = pl.BlockSpec((1, din), lambda i: (0, 0))
    return pl.pallas_call(
        _mid_body,
        grid=(_GRID,),
        in_specs=[pl.BlockSpec((_BM, din), lambda i: (i, 0)),
                  pl.BlockSpec((_BM, din), lambda i: (i, 0)),
                  gspec, gspec,
                  wspec, bspec, wspec, bspec, wspec, bspec, wspec, bspec],
        out_specs=[pl.BlockSpec((_BM, dout), lambda i: (i, 0))] * 4,
        out_shape=[jax.ShapeDtypeStruct((N, dout), jnp.float32)] * 4,
    )(agg, s, g.reshape(1, din), be.reshape(1, din),
      Wq, bq.reshape(1, dout), Wk, bk.reshape(1, dout),
      Wv, bv.reshape(1, dout), Ws, bs.reshape(1, dout))


def _fin_body(agg, s, g, be, o):
    o[...] = _ln_relu(agg[...] + s[...], g[...], be[...])


def _dense_fin(agg, s, g, be):
    d = s.shape[1]
    gspec = pl.BlockSpec((1, d), lambda i: (0, 0))
    return pl.pallas_call(
        _fin_body,
        grid=(_GRID,),
        in_specs=[pl.BlockSpec((_BM, d), lambda i: (i, 0)),
                  pl.BlockSpec((_BM, d), lambda i: (i, 0)), gspec, gspec],
        out_specs=pl.BlockSpec((_BM, d), lambda i: (i, 0)),
        out_shape=jax.ShapeDtypeStruct((N, d), jnp.float32),
    )(agg, s, g.reshape(1, d), be.reshape(1, d))


_CH = 1280


def _sc_compact(src, dst):
    mesh = plsc.VectorSubcoreMesh(**_MESH)

    @functools.partial(
        pl.kernel,
        out_type=(jax.ShapeDtypeStruct((G * CAP,), jnp.int32),
                  jax.ShapeDtypeStruct((G * CAP,), jnp.int32),
                  jax.ShapeDtypeStruct((G * 128,), jnp.int32)),
        mesh=mesh,
        compiler_params=pltpu.CompilerParams(needs_layout_passes=False),
        scratch_types=[
            pltpu.VMEM((_CH,), jnp.int32),
            pltpu.VMEM((_CH,), jnp.int32),
            pltpu.VMEM((CAP,), jnp.int32),
            pltpu.VMEM((CAP,), jnp.int32),
            pltpu.VMEM((CAP,), jnp.int32),
            pltpu.VMEM((CAP,), jnp.int32),
            pltpu.VMEM((CAP,), jnp.int32),
            pltpu.VMEM((CAP,), jnp.int32),
            pltpu.VMEM((128,), jnp.int32),
        ],
    )
    def kern(src_h, dst_h, os_h, od_h, cnt_h, sb, db,
             ls0, ls1, ls2, ld0, ld1, ld2, cb):
        ls = [ls0, ls1, ls2]
        ld = [ld0, ld1, ld2]
        wid = _wid()
        lane = lax.broadcasted_iota(jnp.int32, (16,), 0)
        zero16 = jnp.zeros((16,), jnp.int32)

        def zbody(i, _):
            for g_i in range(GPW):
                ls[g_i][pl.ds(i * 16, 16)] = zero16
                ld[g_i][pl.ds(i * 16, 16)] = zero16
            return 0
        lax.fori_loop(0, CAP // 16, zbody, 0)

        glo0 = wid * GPW * GN

        def chunk(ci, curs):
            base = ci * _CH
            pltpu.sync_copy(src_h.at[pl.ds(base, _CH)], sb)
            pltpu.sync_copy(dst_h.at[pl.ds(base, _CH)], db)

            @plsc.parallel_loop(0, _CH // 16, 1, unroll=2, carry=curs)
            def step(i, curs):
                sv = sb[pl.ds(i * 16, 16)]
                dv = db[pl.ds(i * 16, 16)]
                out = []
                for g_i in range(GPW):
                    lo = glo0 + g_i * GN
                    m = (dv >= lo) & (dv < lo + GN)
                    pos = curs[g_i] + plsc.cumsum(jnp.where(m, 1, 0)) - 1
                    mm = m & (pos < CAP)
                    plsc.store_scatter(ls[g_i], [pos], sv, mask=mm)
                    plsc.store_scatter(ld[g_i], [pos], dv, mask=mm)
                    out.append(curs[g_i] + plsc.all_reduce_population_count(m))
                return tuple(out)

            return step

        curs = lax.fori_loop(0, E // _CH, chunk,
                             tuple(zero16 for _ in range(GPW)))

        for g_i in range(GPW):
            grp = wid * GPW + g_i
            pltpu.sync_copy(ls[g_i], os_h.at[pl.ds(grp * CAP, CAP)])
            pltpu.sync_copy(ld[g_i], od_h.at[pl.ds(grp * CAP, CAP)])
            cb[pl.ds(0, 16)] = curs[g_i]
            pltpu.sync_copy(cb, cnt_h.at[pl.ds(grp * 128, 128)])

    return kern(src, dst)


def _sc_edge_attn(q, k, osrc, odst, cnts, H, C):
    D = H * C
    mesh = plsc.VectorSubcoreMesh(**_MESH)
    inv = 1.0 / math.sqrt(C)

    @functools.partial(
        pl.kernel,
        out_type=jax.ShapeDtypeStruct((G * CAP, 16), jnp.float32),
        mesh=mesh,
        compiler_params=pltpu.CompilerParams(needs_layout_passes=False),
        scratch_types=[
            pltpu.VMEM((CAP + 32,), jnp.int32),
            pltpu.VMEM((CAP + 32,), jnp.int32),
            pltpu.VMEM((128,), jnp.int32),
            pltpu.VMEM((GN, D), jnp.float32),
            pltpu.VMEM((BA, D), jnp.float32),
            pltpu.VMEM((BA, D), jnp.float32),
            pltpu.VMEM((BA, 16), jnp.float32),
            pltpu.SemaphoreType.DMA,
            pltpu.SemaphoreType.DMA,
        ],
    )
    def kern(q_h, k_h, os_h, od_h, cnt_h, ea_h, ls, ld, cb,
             qg, kb0, kb1, eab, smk0, smk1):
        wid = _wid()
        lane = lax.broadcasted_iota(jnp.int32, (16,), 0)
        zi = jnp.zeros((16,), jnp.int32)
        for t in range(2):
            ls[pl.ds(CAP + t * 16, 16)] = zi
            ld[pl.ds(CAP + t * 16, 16)] = zi
        perms = [lane ^ (1 << t) for t in range(4)]
        bufs = [(kb0, smk0), (kb1, smk1)]

        for g_i in range(GPW):
            grp = wid * GPW + g_i
            glo = grp * GN
            pltpu.sync_copy(os_h.at[pl.ds(grp * CAP, CAP)],
                            ls.at[pl.ds(0, CAP)])
            pltpu.sync_copy(od_h.at[pl.ds(grp * CAP, CAP)],
                            ld.at[pl.ds(0, CAP)])
            pltpu.sync_copy(cnt_h.at[pl.ds(grp * 128, 128)], cb)
            pltpu.sync_copy(q_h.at[pl.ds(glo, GN)], qg)
            cnt = jnp.minimum(cb[pl.ds(0, 16)][0], CAP)
            nch = (cnt + BA - 1) // BA

            def issue(j, b):
                kb, smk = bufs[b]
                pltpu.async_copy(k_h.at[ls.at[pl.ds(j * BA, BA)]], kb, smk)

            @pl.when(nch > 0)
            def _():
                issue(0, 0)

            def duo(jo, _):
                for b in range(2):
                    j = 2 * jo + b

                    @pl.when(j < nch)
                    def _():
                        @pl.when(j + 1 < nch)
                        def _():
                            issue(j + 1, 1 - b)
                        kb, smk = bufs[b]
                        pltpu.make_async_copy(
                            k_h.at[pl.ds(0, BA)], kb, smk).wait()
                        dvl0 = ld[pl.ds(j * BA, 16)]
                        dvl1 = ld[pl.ds(j * BA + 8, 16)]

                        @plsc.parallel_loop(0, BA, 1, unroll=2)
                        def edge(e):
                            lsp = jnp.broadcast_to(e & 7, (16,)) \
                                .astype(jnp.int32)
                            dsel = jnp.where(e < 16, _dyn_gather(dvl0, lsp + (e & 8)),
                                             _dyn_gather(dvl1, lsp + 8))
                            r0 = dsel[0] - glo
                            r = jnp.clip(r0, 0, GN - 1)
                            al = jnp.zeros((16,), jnp.float32)
                            for h in range(H):
                                acc = jnp.zeros((16,), jnp.float32)
                                for i in range(C // 16):
                                    off = h * C + i * 16
                                    acc = acc + (qg[r, pl.ds(off, 16)] *
                                                 kb[e, pl.ds(off, 16)])
                                for t in range(4):
                                    acc = acc + _dyn_gather(acc, perms[t])
                                al = jnp.where(lane == h, acc, al)
                            eab[e, :] = jnp.exp(al * inv)

                        pltpu.sync_copy(
                            eab, ea_h.at[pl.ds(grp * CAP + j * BA, BA)])
                return 0

            lax.fori_loop(0, (nch + 1) // 2, duo, 0)

    return kern(q, k, osrc, odst, cnts)


def _sc_edge_agg(v, ea, osrc, odst, cnts, H, C):
    D = H * C
    NV = D // 16
    mesh = plsc.VectorSubcoreMesh(**_MESH)

    @functools.partial(
        pl.kernel,
        out_type=jax.ShapeDtypeStruct((NPAD, D), jnp.float32),
        mesh=mesh,
        compiler_params=pltpu.CompilerParams(needs_layout_passes=False),
        scratch_types=[
            pltpu.VMEM((CAP + 32,), jnp.int32),
            pltpu.VMEM((CAP + 32,), jnp.int32),
            pltpu.VMEM((128,), jnp.int32),
            pltpu.VMEM((BV, D), jnp.float32),
            pltpu.VMEM((BV, D), jnp.float32),
            pltpu.VMEM((BV, 16), jnp.float32),
            pltpu.VMEM((BV, 16), jnp.float32),
            pltpu.VMEM((GN, D), jnp.float32),
            pltpu.VMEM((GN, 16), jnp.float32),
            pltpu.SemaphoreType.DMA,
            pltpu.SemaphoreType.DMA,
            pltpu.SemaphoreType.DMA,
            pltpu.SemaphoreType.DMA,
        ],
    )
    def kern(v_h, ea_h, os_h, od_h, cnt_h, agg_h,
             ls, ld, cb, vb0, vb1, eb0, eb1, acc, den,
             smv0, smv1, sme0, sme1):
        bufs = [(vb0, eb0, smv0, sme0), (vb1, eb1, smv1, sme1)]
        wid = _wid()
        zi = jnp.zeros((16,), jnp.int32)
        for t in range(2):
            ls[pl.ds(CAP + t * 16, 16)] = zi
            ld[pl.ds(CAP + t * 16, 16)] = zi
        lane = lax.broadcasted_iota(jnp.int32, (16,), 0)
        hmask = jnp.where(lane < H, 1.0, 0.0).astype(jnp.float32)
        zerov = jnp.zeros((16,), jnp.float32)
        hidx = [jnp.full((16,), h, jnp.int32) for h in range(H)]

        for g_i in range(GPW):
            grp = wid * GPW + g_i
            glo = grp * GN
            pltpu.sync_copy(os_h.at[pl.ds(grp * CAP, CAP)],
                            ls.at[pl.ds(0, CAP)])
            pltpu.sync_copy(od_h.at[pl.ds(grp * CAP, CAP)],
                            ld.at[pl.ds(0, CAP)])
            pltpu.sync_copy(cnt_h.at[pl.ds(grp * 128, 128)], cb)
            cnt = jnp.minimum(cb[pl.ds(0, 16)][0], CAP)

            @plsc.parallel_loop(0, GN, 1, unroll=1)
            def zbody(r):
                for i in range(NV):
                    acc[r, pl.ds(i * 16, 16)] = zerov
                den[r, :] = zerov

            nch = (cnt + BV - 1) // BV

            def issue(j, b):
                vb, eb, smv, sme = bufs[b]
                pltpu.async_copy(v_h.at[ls.at[pl.ds(j * BV, BV)]], vb, smv)
                pltpu.async_copy(ea_h.at[pl.ds(grp * CAP + j * BV, BV)],
                                 eb, sme)

            @pl.when(nch > 0)
            def _():
                issue(0, 0)

            def duo(jo, _):
                for pb in range(2):
                    ci = 2 * jo + pb

                    @pl.when(ci < nch)
                    def _():
                        @pl.when(ci + 1 < nch)
                        def _():
                            issue(ci + 1, 1 - pb)
                        vb, eb, smv, sme = bufs[pb]
                        pltpu.make_async_copy(
                            v_h.at[pl.ds(0, BV)], vb, smv).wait()
                        pltpu.make_async_copy(
                            ea_h.at[pl.ds(0, BV)], eb, sme).wait()
                        b = ci * BV
                        nn = jnp.minimum(BV, cnt - b)
                        dv0 = ld[pl.ds(b, 16)]
                        dv1 = ld[pl.ds(b + 8, 16)]

                        @plsc.parallel_loop(0, nn, 1, unroll=1)
                        def edge(e):
                            lsp = jnp.broadcast_to(e & 7, (16,)) \
                                .astype(jnp.int32)
                            dsel = jnp.where(
                                e < 16, _dyn_gather(dv0, lsp + (e & 8)),
                                _dyn_gather(dv1, lsp + 8))
                            r = dsel[0] - glo
                            eav = eb[e, :]
                            plsc.addupdate(den.at[r], eav * hmask)
                            esp = jnp.broadcast_to(e, (16,)).astype(jnp.int32)
                            for h in range(H):
                                ehv = plsc.load_gather(eb, [esp, hidx[h]])
                                for i in range(C // 16):
                                    off = h * C + i * 16
                                    plsc.addupdate(
                                        acc.at[r, pl.ds(off, 16)],
                                        ehv * vb[e, pl.ds(off, 16)])
                return 0

            lax.fori_loop(0, (nch + 1) // 2, duo, 0)

            @plsc.parallel_loop(0, GN, 1, unroll=1)
            def fin(r):
                rv = 1.0 / (den[r, :] + 1e-16)
                for h in range(H):
                    rhv = _dyn_gather(rv, hidx[h])
                    for i in range(C // 16):
                        off = h * C + i * 16
                        acc[r, pl.ds(off, 16)] = acc[r, pl.ds(off, 16)] * rhv

            pltpu.sync_copy(acc, agg_h.at[pl.ds(glo, GN)])

    return kern(v, ea, osrc, odst, cnts)


def _sc_edge_fused2(q, k, v, osrc, odst, cnts):
    D = OUT
    NV = D // 16
    mesh = plsc.VectorSubcoreMesh(**_MESH)
    inv = 1.0 / math.sqrt(D)

    @functools.partial(
        pl.kernel,
        out_type=jax.ShapeDtypeStruct((NPAD, D), jnp.float32),
        mesh=mesh,
        compiler_params=pltpu.CompilerParams(needs_layout_passes=False),
        scratch_types=[
            pltpu.VMEM((CAP + 32,), jnp.int32),
            pltpu.VMEM((CAP + 32,), jnp.int32),
            pltpu.VMEM((128,), jnp.int32),
            pltpu.VMEM((GN, D), jnp.float32),
            pltpu.VMEM((BV, D), jnp.float32),
            pltpu.VMEM((BV, D), jnp.float32),
            pltpu.VMEM((BV, D), jnp.float32),
            pltpu.VMEM((BV, D), jnp.float32),
            pltpu.VMEM((GN, D), jnp.float32),
            pltpu.VMEM((GN, 16), jnp.float32),
            pltpu.SemaphoreType.DMA,
            pltpu.SemaphoreType.DMA,
            pltpu.SemaphoreType.DMA,
            pltpu.SemaphoreType.DMA,
        ],
    )
    def kern(q_h, k_h, v_h, os_h, od_h, cnt_h, agg_h,
             ls, ld, cb, qg, kb0, kb1, vb0, vb1, acc, den,
             smk0, smk1, smv0, smv1):
        wid = _wid()
        lane = lax.broadcasted_iota(jnp.int32, (16,), 0)
        perms = [lane ^ (1 << t) for t in range(4)]
        hmask = jnp.where(lane < 1, 1.0, 0.0).astype(jnp.float32)
        zerov = jnp.zeros((16,), jnp.float32)
        zidx = jnp.zeros((16,), jnp.int32)
        zi = jnp.zeros((16,), jnp.int32)
        for t in range(2):
            ls[pl.ds(CAP + t * 16, 16)] = zi
            ld[pl.ds(CAP + t * 16, 16)] = zi
        bufs = [(kb0, vb0, smk0, smv0), (kb1, vb1, smk1, smv1)]

        for g_i in range(GPW):
            grp = wid * GPW + g_i
            glo = grp * GN
            pltpu.sync_copy(os_h.at[pl.ds(grp * CAP, CAP)],
                            ls.at[pl.ds(0, CAP)])
            pltpu.sync_copy(od_h.at[pl.ds(grp * CAP, CAP)],
                            ld.at[pl.ds(0, CAP)])
            pltpu.sync_copy(cnt_h.at[pl.ds(grp * 128, 128)], cb)
            pltpu.sync_copy(q_h.at[pl.ds(glo, GN)], qg)
            cnt = jnp.minimum(cb[pl.ds(0, 16)][0], CAP)

            @plsc.parallel_loop(0, GN, 1, unroll=1)
            def zbody(r):
                for i in range(NV):
                    acc[r, pl.ds(i * 16, 16)] = zerov
                den[r, :] = zerov

            nch = (cnt + BV - 1) // BV

            def issue(j, b):
                kb, vb, smk, smv = bufs[b]
                pltpu.async_copy(k_h.at[ls.at[pl.ds(j * BV, BV)]], kb, smk)
                pltpu.async_copy(v_h.at[ls.at[pl.ds(j * BV, BV)]], vb, smv)

            @pl.when(nch > 0)
            def _():
                issue(0, 0)

            def duo(jo, _):
                for pb in range(2):
                    ci = 2 * jo + pb

                    @pl.when(ci < nch)
                    def _():
                        @pl.when(ci + 1 < nch)
                        def _():
                            issue(ci + 1, 1 - pb)
                        kb, vb, smk, smv = bufs[pb]
                        pltpu.make_async_copy(
                            k_h.at[pl.ds(0, BV)], kb, smk).wait()
                        pltpu.make_async_copy(
                            v_h.at[pl.ds(0, BV)], vb, smv).wait()
                        b = ci * BV
                        nn = jnp.minimum(BV, cnt - b)
                        dv0 = ld[pl.ds(b, 16)]
                        dv1 = ld[pl.ds(b + 8, 16)]

                        @plsc.parallel_loop(0, nn, 1, unroll=1)
                        def edge(e):
                            lsp = jnp.broadcast_to(e & 7, (16,)) \
                                .astype(jnp.int32)
                            dsel = jnp.where(
                                e < 16, _dyn_gather(dv0, lsp + (e & 8)),
                                _dyn_gather(dv1, lsp + 8))
                            r = dsel[0] - glo
                            a = jnp.zeros((16,), jnp.float32)
                            for i in range(NV):
                                off = i * 16
                                a = a + (qg[r, pl.ds(off, 16)] *
                                         kb[e, pl.ds(off, 16)])
                            for t in range(4):
                                a = a + _dyn_gather(a, perms[t])
                            ea = jnp.exp(a * inv)
                            plsc.addupdate(den.at[r], ea * hmask)
                            for i in range(NV):
                                off = i * 16
                                plsc.addupdate(acc.at[r, pl.ds(off, 16)],
                                               ea * vb[e, pl.ds(off, 16)])

                return 0

            lax.fori_loop(0, (nch + 1) // 2, duo, 0)

            @plsc.parallel_loop(0, GN, 1, unroll=1)
            def fin(r):
                rv = 1.0 / (den[r, :] + 1e-16)
                rhv = _dyn_gather(rv, zidx)
                for i in range(NV):
                    off = i * 16
                    acc[r, pl.ds(off, 16)] = acc[r, pl.ds(off, 16)] * rhv

            pltpu.sync_copy(acc, agg_h.at[pl.ds(glo, GN)])

    return kern(q, k, v, osrc, odst, cnts)


def kernel(x, edge_index, Wq1, bq1, Wk1, bk1, Wv1, bv1, Ws1, bs1, g1, be1,
           Wq2, bq2, Wk2, bk2, Wv2, bv2, Ws2, bs2, g2, be2):
    src = edge_index[0].astype(jnp.int32)
    dst = edge_index[1].astype(jnp.int32)

    q1, k1, v1, s1 = _dense_proj(x, Wq1, bq1, Wk1, bk1, Wv1, bv1, Ws1, bs1)
    osrc, odst, cnts = _sc_compact(src, dst)
    ea1 = _sc_edge_attn(q1, k1, osrc, odst, cnts, HEADS, HID)
    agg1 = _sc_edge_agg(v1, ea1, osrc, odst, cnts, HEADS, HID)

    q2, k2, v2, s2 = _dense_mid(agg1[:N], s1, g1, be1,
                                Wq2, bq2, Wk2, bk2, Wv2, bv2, Ws2, bs2)
    agg2 = _sc_edge_fused2(q2, k2, v2, osrc, odst, cnts)

    return _dense_fin(agg2[:N], s2, g2, be2)
